# baseline (speedup 1.0000x reference)
import functools

import jax
import jax.numpy as jnp
from jax import lax
from jax.experimental import pallas as pl
from jax.experimental.pallas import tpu as pltpu
from jax.experimental.pallas import tpu_sc as plsc

B = 16384
D = 32
PACK = 128 // D
NC = 2
NS = 16
NW = NC * NS
BPW = B // NW
NCHUNK = 4
CHUNK = BPW // NCHUNK
NGRP = CHUNK // 16


def _fire_chunk(ut_hbm, it_hbm, gidx_u, gidx_v, buf_u, buf_v, sem, j):
    slot = j % 2
    cu = pltpu.async_copy(ut_hbm.at[gidx_u.at[j]], buf_u.at[slot], sem)
    cv = pltpu.async_copy(it_hbm.at[gidx_v.at[j]], buf_v.at[slot], sem)
    return cu, cv


def _body(user_hbm, item_hbm, ut_hbm, it_hbm, out_hbm,
          oidx_u, oidx_v, gidx_u, gidx_v, buf_u, buf_v, out_v,
          sem_idx, sem_rows):
    wid = lax.axis_index("s") * NC + lax.axis_index("c")
    base = wid * BPW

    idx_cps = []
    for j in range(NCHUNK):
        idx_cps.append(pltpu.async_copy(
            user_hbm.at[pl.ds(base + j * CHUNK, CHUNK)], oidx_u.at[j], sem_idx))
        idx_cps.append(pltpu.async_copy(
            item_hbm.at[pl.ds(base + j * CHUNK, CHUNK)], oidx_v.at[j], sem_idx))
    for cp in idx_cps:
        cp.wait()

    def shift(t, carry):
        for c in range(NCHUNK):
            gidx_u[c, pl.ds(t * 16, 16)] = oidx_u[c, pl.ds(t * 16, 16)] // PACK
            gidx_v[c, pl.ds(t * 16, 16)] = oidx_v[c, pl.ds(t * 16, 16)] // PACK
        return carry

    lax.fori_loop(0, NGRP, shift, 0)

    lane = lax.iota(jnp.int32, 16)
    q_mask = jnp.full((16,), PACK - 1, jnp.int32)

    cps = _fire_chunk(ut_hbm, it_hbm, gidx_u, gidx_v, buf_u, buf_v,
                      sem_rows, 0)
    for j in range(NCHUNK):
        for cp in cps:
            cp.wait()
        if j + 1 < NCHUNK:
            cps = _fire_chunk(ut_hbm, it_hbm, gidx_u, gidx_v, buf_u, buf_v,
                              sem_rows, j + 1)
        bu = buf_u.at[j % 2]
        bv = buf_v.at[j % 2]

        def group(t, carry, j=j, bu=bu, bv=bv):
            r = lane + t * 16
            qu = (oidx_u[j, pl.ds(t * 16, 16)] & q_mask) * D
            qv = (oidx_v[j, pl.ds(t * 16, 16)] & q_mask) * D
            acc = jnp.zeros((16,), jnp.float32)
            for d in range(D):
                u = plsc.load_gather(bu, [r, qu + d])
                v = plsc.load_gather(bv, [r, qv + d])
                acc = acc + u * v
            out_v[pl.ds(j * CHUNK + t * 16, 16)] = acc
            return carry

        lax.fori_loop(0, NGRP, group, 0)

    pltpu.sync_copy(out_v, out_hbm.at[pl.ds(base, BPW)])


@functools.partial(
    pl.kernel,
    out_type=jax.ShapeDtypeStruct((B,), jnp.float32),
    mesh=plsc.VectorSubcoreMesh(core_axis_name="c", subcore_axis_name="s"),
    compiler_params=pltpu.CompilerParams(needs_layout_passes=False),
    scratch_types=[
        pltpu.VMEM((NCHUNK, CHUNK), jnp.int32),
        pltpu.VMEM((NCHUNK, CHUNK), jnp.int32),
        pltpu.VMEM((NCHUNK, CHUNK), jnp.int32),
        pltpu.VMEM((NCHUNK, CHUNK), jnp.int32),
        pltpu.VMEM((2, CHUNK, 128), jnp.float32),
        pltpu.VMEM((2, CHUNK, 128), jnp.float32),
        pltpu.VMEM((BPW,), jnp.float32),
        pltpu.SemaphoreType.DMA,
        pltpu.SemaphoreType.DMA,
    ],
)
def _dot_kernel(user_hbm, item_hbm, ut_hbm, it_hbm, out_hbm,
                oidx_u, oidx_v, gidx_u, gidx_v, buf_u, buf_v, out_v,
                sem_idx, sem_rows):
    _body(user_hbm, item_hbm, ut_hbm, it_hbm, out_hbm,
          oidx_u, oidx_v, gidx_u, gidx_v, buf_u, buf_v, out_v,
          sem_idx, sem_rows)


def kernel(user, item, user_table, item_table):
    return _dot_kernel(user.astype(jnp.int32), item.astype(jnp.int32),
                       user_table.reshape(-1, 128), item_table.reshape(-1, 128))


# speedup vs baseline: 4.5849x; 4.5849x over previous
import functools

import jax
import jax.numpy as jnp
from jax import lax
from jax.experimental import pallas as pl
from jax.experimental.pallas import tpu as pltpu
from jax.experimental.pallas import tpu_sc as plsc

B = 16384
D = 32
NC = 2
NS = 16
NW = NC * NS
BPW = B // NW
NCHUNK = 4
CHUNK = BPW // NCHUNK
VPC = CHUNK // 16
SLOTS = 8
NGRP = BPW // 16


def _body(user_hbm, item_hbm, ut_hbm, it_hbm, out_hbm,
          oidx_u, oidx_v, buf_u, buf_v, out_v, sem_idx, sem_col):
    wid = lax.axis_index("s") * NC + lax.axis_index("c")
    base = wid * BPW

    idx_cps = []
    for j in range(NCHUNK):
        idx_cps.append(pltpu.async_copy(
            user_hbm.at[pl.ds(base + j * CHUNK, CHUNK)], oidx_u.at[j], sem_idx))
        idx_cps.append(pltpu.async_copy(
            item_hbm.at[pl.ds(base + j * CHUNK, CHUNK)], oidx_v.at[j], sem_idx))
    for cp in idx_cps:
        cp.wait()

    lane = lax.iota(jnp.int32, 16)

    def idx_vecs(g):
        j = g // VPC
        off = (g % VPC) * 16
        return oidx_u[j, pl.ds(off, 16)], oidx_v[j, pl.ds(off, 16)]

    def extract(vec, i):
        return jnp.sum(jnp.where(lane == i, vec, 0))

    def fire(cu, cv, s):
        cu_blk = pl.multiple_of((cu >> 7) * 128, 128)
        cv_blk = pl.multiple_of((cv >> 7) * 128, 128)
        pltpu.async_copy(
            ut_hbm.at[:, pl.ds(cu_blk, 128)], buf_u.at[s], sem_col)
        pltpu.async_copy(
            it_hbm.at[:, pl.ds(cv_blk, 128)], buf_v.at[s], sem_col)

    iu0, iv0 = idx_vecs(0)
    for i in range(SLOTS):
        fire(extract(iu0, i), extract(iv0, i), i)

    def group(g, carry):
        iu, iv = idx_vecs(g)
        iu_next, iv_next = idx_vecs((g + 1) % NGRP)
        acc = jnp.zeros((16,), jnp.float32)
        for i in range(16):
            r = g * 16 + i
            s = r % SLOTS
            pltpu.make_async_copy(
                ut_hbm.at[:, pl.ds(0, 128)], buf_u.at[s], sem_col).wait()
            pltpu.make_async_copy(
                it_hbm.at[:, pl.ds(0, 128)], buf_v.at[s], sem_col).wait()
            cu = extract(iu, i)
            cv = extract(iv, i)
            cu_lo = jnp.full((16,), cu & 127, jnp.int32)
            cv_lo = jnp.full((16,), cv & 127, jnp.int32)
            svec = jnp.full((16,), s, jnp.int32)
            u0 = plsc.load_gather(buf_u, [svec, lane, cu_lo])
            u1 = plsc.load_gather(buf_u, [svec, lane + 16, cu_lo])
            v0 = plsc.load_gather(buf_v, [svec, lane, cv_lo])
            v1 = plsc.load_gather(buf_v, [svec, lane + 16, cv_lo])
            total = jnp.sum(u0 * v0 + u1 * v1)
            acc = jnp.where(lane == i, total, acc)

            @pl.when(r + SLOTS < BPW)
            def _():
                if i < 16 - SLOTS:
                    cu_n = extract(iu, i + SLOTS)
                    cv_n = extract(iv, i + SLOTS)
                else:
                    cu_n = extract(iu_next, i + SLOTS - 16)
                    cv_n = extract(iv_next, i + SLOTS - 16)
                fire(cu_n, cv_n, s)

        out_v[pl.ds(g * 16, 16)] = acc
        return carry

    lax.fori_loop(0, NGRP, group, 0)
    pltpu.sync_copy(out_v, out_hbm.at[pl.ds(base, BPW)])


@functools.partial(
    pl.kernel,
    out_type=jax.ShapeDtypeStruct((B,), jnp.float32),
    mesh=plsc.VectorSubcoreMesh(core_axis_name="c", subcore_axis_name="s"),
    compiler_params=pltpu.CompilerParams(needs_layout_passes=False),
    scratch_types=[
        pltpu.VMEM((NCHUNK, CHUNK), jnp.int32),
        pltpu.VMEM((NCHUNK, CHUNK), jnp.int32),
        pltpu.VMEM((SLOTS, D, 128), jnp.float32),
        pltpu.VMEM((SLOTS, D, 128), jnp.float32),
        pltpu.VMEM((BPW,), jnp.float32),
        pltpu.SemaphoreType.DMA,
        pltpu.SemaphoreType.DMA,
    ],
)
def _dot_kernel(user_hbm, item_hbm, ut_hbm, it_hbm, out_hbm,
                oidx_u, oidx_v, buf_u, buf_v, out_v, sem_idx, sem_col):
    _body(user_hbm, item_hbm, ut_hbm, it_hbm, out_hbm,
          oidx_u, oidx_v, buf_u, buf_v, out_v, sem_idx, sem_col)


def kernel(user, item, user_table, item_table):
    return _dot_kernel(user.astype(jnp.int32), item.astype(jnp.int32),
                       user_table.T, item_table.T)
